# baseline (speedup 1.0000x reference)
import jax
import jax.numpy as jnp
from jax import lax
from jax.experimental import pallas as pl
from jax.experimental.pallas import tpu as pltpu
from jax.experimental.pallas import tpu_sc as plsc

NC = 2
NS = 16
NW = NC * NS
IB = 128
CB = 32
ZB = 1600


def _edge_pass(np_, epw_rows, n_chunks):
    npw = np_ // NS
    mesh = plsc.VectorSubcoreMesh(core_axis_name="c", subcore_axis_name="s")

    def body(g0_hbm, g1_hbm, g2_hbm, src_hbm, dst_hbm,
             o0_hbm, o1_hbm, o2_hbm,
             t0, t1, t2, a0, a1, a2, src_v, dst_v,
             r0, r1, r2, q0, q1, q2, z_v, sem, ssem):
        c = lax.axis_index("c")
        s = lax.axis_index("s")
        w = c * NS + s
        sl = pl.ds(s * npw, npw)
        pltpu.sync_copy(g0_hbm.at[sl], t0.at[sl])
        pltpu.sync_copy(g1_hbm.at[sl], t1.at[sl])
        pltpu.sync_copy(g2_hbm.at[sl], t2.at[sl])

        def zfill(j, carry):
            z_v[pl.ds(j * 16, 16)] = jnp.zeros((16,), jnp.float32)
            return carry

        lax.fori_loop(0, ZB // 16, zfill, 0)

        def zcopy(j, carry):
            zsl = pl.ds(s * npw + j * ZB, ZB)
            pltpu.sync_copy(z_v, a0.at[zsl])
            pltpu.sync_copy(z_v, a1.at[zsl])
            pltpu.sync_copy(z_v, a2.at[zsl])
            return carry

        lax.fori_loop(0, npw // ZB, zcopy, 0)
        plsc.subcore_barrier()

        def outer(o, carry):
            e0 = (w * n_chunks + o) * CB * IB
            pltpu.sync_copy(src_hbm.at[pl.ds(e0, CB * IB)], src_v)
            pltpu.sync_copy(dst_hbm.at[pl.ds(e0, CB * IB)], dst_v)

            rbufs = ((r0, r1, r2), (q0, q1, q2))

            def gath(j):
                si = src_v.at[pl.ds(j * IB, IB)]
                b = rbufs[j % 2]
                return (pltpu.async_copy(t0.at[si], b[0], sem),
                        pltpu.async_copy(t1.at[si], b[1], sem),
                        pltpu.async_copy(t2.at[si], b[2], sem))

            def scat(j):
                di = dst_v.at[pl.ds(j * IB, IB)]
                b = rbufs[j % 2]
                return (pltpu.async_copy(b[0], a0.at[di], ssem, add=True),
                        pltpu.async_copy(b[1], a1.at[di], ssem, add=True),
                        pltpu.async_copy(b[2], a2.at[di], ssem, add=True))

            g_prev = gath(0)
            s_prev = None
            for j in range(CB):
                if s_prev is not None:
                    for cp in s_prev:
                        cp.wait()
                g_next = gath(j + 1) if j + 1 < CB else None
                for cp in g_prev:
                    cp.wait()
                s_prev = scat(j)
                g_prev = g_next
            for cp in s_prev:
                cp.wait()
            return carry

        lax.fori_loop(0, n_chunks, outer, 0)

        plsc.subcore_barrier()
        osl = pl.ds(c * np_ + s * npw, npw)
        pltpu.sync_copy(a0.at[sl], o0_hbm.at[osl])
        pltpu.sync_copy(a1.at[sl], o1_hbm.at[osl])
        pltpu.sync_copy(a2.at[sl], o2_hbm.at[osl])

    col = jax.ShapeDtypeStruct((NC * np_,), jnp.float32)
    return pl.kernel(
        body,
        out_type=(col, col, col),
        mesh=mesh,
        scratch_types=[
            pltpu.VMEM_SHARED((np_,), jnp.float32),
            pltpu.VMEM_SHARED((np_,), jnp.float32),
            pltpu.VMEM_SHARED((np_,), jnp.float32),
            pltpu.VMEM_SHARED((np_,), jnp.float32),
            pltpu.VMEM_SHARED((np_,), jnp.float32),
            pltpu.VMEM_SHARED((np_,), jnp.float32),
            pltpu.VMEM((CB * IB,), jnp.int32),
            pltpu.VMEM((CB * IB,), jnp.int32),
            pltpu.VMEM((IB,), jnp.float32),
            pltpu.VMEM((IB,), jnp.float32),
            pltpu.VMEM((IB,), jnp.float32),
            pltpu.VMEM((IB,), jnp.float32),
            pltpu.VMEM((IB,), jnp.float32),
            pltpu.VMEM((IB,), jnp.float32),
            pltpu.VMEM((ZB,), jnp.float32),
            pltpu.SemaphoreType.DMA,
            pltpu.SemaphoreType.DMA,
        ],
    )


def _deg_pass(np_, epw_rows, n_chunks):
    npw = np_ // NS
    mesh = plsc.VectorSubcoreMesh(core_axis_name="c", subcore_axis_name="s")

    def body(dst_hbm, o0_hbm, a0, dst_v, r0, z_v):
        c = lax.axis_index("c")
        s = lax.axis_index("s")
        w = c * NS + s
        sl = pl.ds(s * npw, npw)

        def zfill(j, carry):
            z_v[pl.ds(j * 16, 16)] = jnp.zeros((16,), jnp.float32)
            return carry

        lax.fori_loop(0, ZB // 16, zfill, 0)

        def onesfill(j, carry):
            r0[pl.ds(j * 16, 16)] = jnp.ones((16,), jnp.float32)
            return carry

        lax.fori_loop(0, IB // 16, onesfill, 0)

        def zcopy(j, carry):
            pltpu.sync_copy(z_v, a0.at[pl.ds(s * npw + j * ZB, ZB)])
            return carry

        lax.fori_loop(0, npw // ZB, zcopy, 0)
        plsc.subcore_barrier()

        def outer(o, carry):
            e0 = (w * n_chunks + o) * CB * IB
            pltpu.sync_copy(dst_hbm.at[pl.ds(e0, CB * IB)], dst_v)

            def inner(j, carry2):
                pltpu.sync_copy(r0, a0.at[dst_v.at[pl.ds(j * IB, IB)]], add=True)
                return carry2

            return lax.fori_loop(0, CB, inner, carry)

        lax.fori_loop(0, n_chunks, outer, 0)

        plsc.subcore_barrier()
        pltpu.sync_copy(a0.at[sl], o0_hbm.at[pl.ds(c * np_ + s * npw, npw)])

    return pl.kernel(
        body,
        out_type=jax.ShapeDtypeStruct((NC * np_,), jnp.float32),
        mesh=mesh,
        scratch_types=[
            pltpu.VMEM_SHARED((np_,), jnp.float32),
            pltpu.VMEM((CB * IB,), jnp.int32),
            pltpu.VMEM((IB,), jnp.float32),
            pltpu.VMEM((ZB,), jnp.float32),
        ],
    )


def _f(col, np_):
    return col.reshape(np_ // 128, 128)


def _tc_dinv(d, np_):
    def body(d_ref, o_ref):
        o_ref[...] = lax.rsqrt(d_ref[0] + d_ref[1] + 1.0)
    return pl.pallas_call(
        body, out_shape=jax.ShapeDtypeStruct((np_ // 128, 128), jnp.float32))(
            d.reshape(NC, np_ // 128, 128))


def _tc_g0(x0, x1, x2, dinv, w):
    def body(x0_ref, x1_ref, x2_ref, di_ref, w_ref, o0_ref, o1_ref, o2_ref):
        di = di_ref[...]
        for c, o_ref in enumerate((o0_ref, o1_ref, o2_ref)):
            o_ref[...] = di * (x0_ref[...] * w_ref[0, c]
                               + x1_ref[...] * w_ref[1, c]
                               + x2_ref[...] * w_ref[2, c])
    sh = jax.ShapeDtypeStruct(x0.shape, jnp.float32)
    return pl.pallas_call(body, out_shape=(sh, sh, sh))(x0, x1, x2, dinv, w)


def _tc_layer(sp, g, dinv, b, w):
    def body(s0_ref, s1_ref, s2_ref, g0_ref, g1_ref, g2_ref,
             di_ref, b_ref, w_ref, o0_ref, o1_ref, o2_ref):
        di = di_ref[...]
        acts = []
        for c, (s_ref, g_ref) in enumerate(
                zip((s0_ref, s1_ref, s2_ref), (g0_ref, g1_ref, g2_ref))):
            out_c = di * (s_ref[0] + s_ref[1] + g_ref[...]) + b_ref[c]
            acts.append(jnp.maximum(out_c, 0.0))
        for c, o_ref in enumerate((o0_ref, o1_ref, o2_ref)):
            o_ref[...] = di * (acts[0] * w_ref[0, c]
                               + acts[1] * w_ref[1, c]
                               + acts[2] * w_ref[2, c])
    sh = jax.ShapeDtypeStruct(g[0].shape, jnp.float32)
    return pl.pallas_call(body, out_shape=(sh, sh, sh))(
        sp[0], sp[1], sp[2], g[0], g[1], g[2], dinv, b, w)


def _tc_final(sp, g, dinv, b):
    def body(s0_ref, s1_ref, s2_ref, g0_ref, g1_ref, g2_ref,
             di_ref, b_ref, o0_ref, o1_ref, o2_ref):
        di = di_ref[...]
        for c, (s_ref, g_ref, o_ref) in enumerate(
                zip((s0_ref, s1_ref, s2_ref), (g0_ref, g1_ref, g2_ref),
                    (o0_ref, o1_ref, o2_ref))):
            o_ref[...] = di * (s_ref[0] + s_ref[1] + g_ref[...]) + b_ref[c]
    sh = jax.ShapeDtypeStruct(g[0].shape, jnp.float32)
    return pl.pallas_call(body, out_shape=(sh, sh, sh))(
        sp[0], sp[1], sp[2], g[0], g[1], g[2], dinv, b)


def kernel(x, edge_index, W1, b1, W2, b2, W3, b3):
    n, d = x.shape
    e = edge_index.shape[1]

    np_ = -(-n // 25600) * 25600
    nf = np_ // 128

    n_chunks = -(-e // (NW * IB * CB))
    epw_rows = n_chunks * CB
    ep = NW * epw_rows * IB

    pad = jnp.full((2, ep - e), n, dtype=edge_index.dtype)
    ei = jnp.concatenate([edge_index, pad], axis=1)
    src2d = ei[0]
    dst2d = ei[1]

    def colpad(v):
        return jnp.zeros((np_,), jnp.float32).at[:n].set(v).reshape(nf, 128)

    xcols = tuple(colpad(x[:, c]) for c in range(d))

    epass = _edge_pass(np_, epw_rows, n_chunks)
    dpass = _deg_pass(np_, epw_rows, n_chunks)

    def flat3(t):
        return tuple(v.reshape(np_,) for v in t)

    def stack3(t):
        return tuple(v.reshape(NC, nf, 128) for v in t)

    degp = dpass(dst2d)
    dinv = _tc_dinv(degp, np_)

    g = _tc_g0(*xcols, dinv, W1)
    sp = stack3(epass(*flat3(g), src2d, dst2d))
    g = _tc_layer(sp, g, dinv, b1, W2)
    sp = stack3(epass(*flat3(g), src2d, dst2d))
    g = _tc_layer(sp, g, dinv, b2, W3)
    sp = stack3(epass(*flat3(g), src2d, dst2d))
    out = _tc_final(sp, g, dinv, b3)
    return jnp.stack([o.reshape(np_)[:n] for o in out], axis=1)


# speedup vs baseline: 1.5151x; 1.5151x over previous
import jax
import jax.numpy as jnp
from jax import lax
from jax.experimental import pallas as pl
from jax.experimental.pallas import tpu as pltpu
from jax.experimental.pallas import tpu_sc as plsc

NC = 2
NS = 16
NW = NC * NS
IB = 128
CB = 56
ZB = 1600


def _edge_pass(np_, epw_rows, n_chunks):
    npw = np_ // NS
    mesh = plsc.VectorSubcoreMesh(core_axis_name="c", subcore_axis_name="s")

    def body(g0_hbm, g1_hbm, g2_hbm, src_hbm, dst_hbm,
             o0_hbm, o1_hbm, o2_hbm,
             t0, t1, t2, a0, a1, a2, src_v, dst_v,
             r0, r1, r2, q0, q1, q2, z_v, sem, ssem):
        c = lax.axis_index("c")
        s = lax.axis_index("s")
        w = c * NS + s
        sl = pl.ds(s * npw, npw)
        pltpu.sync_copy(g0_hbm.at[sl], t0.at[sl])
        pltpu.sync_copy(g1_hbm.at[sl], t1.at[sl])
        pltpu.sync_copy(g2_hbm.at[sl], t2.at[sl])

        def zfill(j, carry):
            z_v[pl.ds(j * 16, 16)] = jnp.zeros((16,), jnp.float32)
            return carry

        lax.fori_loop(0, ZB // 16, zfill, 0)

        def zcopy(j, carry):
            zsl = pl.ds(s * npw + j * ZB, ZB)
            pltpu.sync_copy(z_v, a0.at[zsl])
            pltpu.sync_copy(z_v, a1.at[zsl])
            pltpu.sync_copy(z_v, a2.at[zsl])
            return carry

        lax.fori_loop(0, npw // ZB, zcopy, 0)
        plsc.subcore_barrier()

        def outer(o, carry):
            e0 = (w * n_chunks + o) * CB * IB
            pltpu.sync_copy(src_hbm.at[pl.ds(e0, CB * IB)], src_v)
            pltpu.sync_copy(dst_hbm.at[pl.ds(e0, CB * IB)], dst_v)

            rbufs = ((r0, r1, r2), (q0, q1, q2))

            def gath(j):
                si = src_v.at[pl.ds(j * IB, IB)]
                b = rbufs[j % 2]
                return (pltpu.async_copy(t0.at[si], b[0], sem),
                        pltpu.async_copy(t1.at[si], b[1], sem),
                        pltpu.async_copy(t2.at[si], b[2], sem))

            def scat(j):
                di = dst_v.at[pl.ds(j * IB, IB)]
                b = rbufs[j % 2]
                return (pltpu.async_copy(b[0], a0.at[di], ssem, add=True),
                        pltpu.async_copy(b[1], a1.at[di], ssem, add=True),
                        pltpu.async_copy(b[2], a2.at[di], ssem, add=True))

            g_prev = gath(0)
            s_prev = None
            for j in range(CB):
                if s_prev is not None:
                    for cp in s_prev:
                        cp.wait()
                g_next = gath(j + 1) if j + 1 < CB else None
                for cp in g_prev:
                    cp.wait()
                s_prev = scat(j)
                g_prev = g_next
            for cp in s_prev:
                cp.wait()
            return carry

        lax.fori_loop(0, n_chunks, outer, 0)

        plsc.subcore_barrier()
        osl = pl.ds(c * np_ + s * npw, npw)
        pltpu.sync_copy(a0.at[sl], o0_hbm.at[osl])
        pltpu.sync_copy(a1.at[sl], o1_hbm.at[osl])
        pltpu.sync_copy(a2.at[sl], o2_hbm.at[osl])

    col = jax.ShapeDtypeStruct((NC * np_,), jnp.float32)
    return pl.kernel(
        body,
        out_type=(col, col, col),
        mesh=mesh,
        scratch_types=[
            pltpu.VMEM_SHARED((np_,), jnp.float32),
            pltpu.VMEM_SHARED((np_,), jnp.float32),
            pltpu.VMEM_SHARED((np_,), jnp.float32),
            pltpu.VMEM_SHARED((np_,), jnp.float32),
            pltpu.VMEM_SHARED((np_,), jnp.float32),
            pltpu.VMEM_SHARED((np_,), jnp.float32),
            pltpu.VMEM((CB * IB,), jnp.int32),
            pltpu.VMEM((CB * IB,), jnp.int32),
            pltpu.VMEM((IB,), jnp.float32),
            pltpu.VMEM((IB,), jnp.float32),
            pltpu.VMEM((IB,), jnp.float32),
            pltpu.VMEM((IB,), jnp.float32),
            pltpu.VMEM((IB,), jnp.float32),
            pltpu.VMEM((IB,), jnp.float32),
            pltpu.VMEM((ZB,), jnp.float32),
            pltpu.SemaphoreType.DMA,
            pltpu.SemaphoreType.DMA,
        ],
    )


def _deg_pass(np_, epw_rows, n_chunks):
    npw = np_ // NS
    mesh = plsc.VectorSubcoreMesh(core_axis_name="c", subcore_axis_name="s")

    def body(dst_hbm, o0_hbm, a0, dst_v, r0, z_v):
        c = lax.axis_index("c")
        s = lax.axis_index("s")
        w = c * NS + s
        sl = pl.ds(s * npw, npw)

        def zfill(j, carry):
            z_v[pl.ds(j * 16, 16)] = jnp.zeros((16,), jnp.float32)
            return carry

        lax.fori_loop(0, ZB // 16, zfill, 0)

        def onesfill(j, carry):
            r0[pl.ds(j * 16, 16)] = jnp.ones((16,), jnp.float32)
            return carry

        lax.fori_loop(0, IB // 16, onesfill, 0)

        def zcopy(j, carry):
            pltpu.sync_copy(z_v, a0.at[pl.ds(s * npw + j * ZB, ZB)])
            return carry

        lax.fori_loop(0, npw // ZB, zcopy, 0)
        plsc.subcore_barrier()

        def outer(o, carry):
            e0 = (w * n_chunks + o) * CB * IB
            pltpu.sync_copy(dst_hbm.at[pl.ds(e0, CB * IB)], dst_v)

            def inner(j, carry2):
                pltpu.sync_copy(r0, a0.at[dst_v.at[pl.ds(j * IB, IB)]], add=True)
                return carry2

            return lax.fori_loop(0, CB, inner, carry)

        lax.fori_loop(0, n_chunks, outer, 0)

        plsc.subcore_barrier()
        pltpu.sync_copy(a0.at[sl], o0_hbm.at[pl.ds(c * np_ + s * npw, npw)])

    return pl.kernel(
        body,
        out_type=jax.ShapeDtypeStruct((NC * np_,), jnp.float32),
        mesh=mesh,
        scratch_types=[
            pltpu.VMEM_SHARED((np_,), jnp.float32),
            pltpu.VMEM((CB * IB,), jnp.int32),
            pltpu.VMEM((IB,), jnp.float32),
            pltpu.VMEM((ZB,), jnp.float32),
        ],
    )


def _f(col, np_):
    return col.reshape(np_ // 128, 128)


def _tc_dinv(d, np_):
    def body(d_ref, o_ref):
        o_ref[...] = lax.rsqrt(d_ref[0] + d_ref[1] + 1.0)
    return pl.pallas_call(
        body, out_shape=jax.ShapeDtypeStruct((np_ // 128, 128), jnp.float32))(
            d.reshape(NC, np_ // 128, 128))


def _tc_g0(x0, x1, x2, dinv, w):
    def body(x0_ref, x1_ref, x2_ref, di_ref, w_ref, o0_ref, o1_ref, o2_ref):
        di = di_ref[...]
        for c, o_ref in enumerate((o0_ref, o1_ref, o2_ref)):
            o_ref[...] = di * (x0_ref[...] * w_ref[0, c]
                               + x1_ref[...] * w_ref[1, c]
                               + x2_ref[...] * w_ref[2, c])
    sh = jax.ShapeDtypeStruct(x0.shape, jnp.float32)
    return pl.pallas_call(body, out_shape=(sh, sh, sh))(x0, x1, x2, dinv, w)


def _tc_layer(sp, g, dinv, b, w):
    def body(s0_ref, s1_ref, s2_ref, g0_ref, g1_ref, g2_ref,
             di_ref, b_ref, w_ref, o0_ref, o1_ref, o2_ref):
        di = di_ref[...]
        acts = []
        for c, (s_ref, g_ref) in enumerate(
                zip((s0_ref, s1_ref, s2_ref), (g0_ref, g1_ref, g2_ref))):
            out_c = di * (s_ref[0] + s_ref[1] + g_ref[...]) + b_ref[c]
            acts.append(jnp.maximum(out_c, 0.0))
        for c, o_ref in enumerate((o0_ref, o1_ref, o2_ref)):
            o_ref[...] = di * (acts[0] * w_ref[0, c]
                               + acts[1] * w_ref[1, c]
                               + acts[2] * w_ref[2, c])
    sh = jax.ShapeDtypeStruct(g[0].shape, jnp.float32)
    return pl.pallas_call(body, out_shape=(sh, sh, sh))(
        sp[0], sp[1], sp[2], g[0], g[1], g[2], dinv, b, w)


def _tc_final(sp, g, dinv, b):
    def body(s0_ref, s1_ref, s2_ref, g0_ref, g1_ref, g2_ref,
             di_ref, b_ref, o0_ref, o1_ref, o2_ref):
        di = di_ref[...]
        for c, (s_ref, g_ref, o_ref) in enumerate(
                zip((s0_ref, s1_ref, s2_ref), (g0_ref, g1_ref, g2_ref),
                    (o0_ref, o1_ref, o2_ref))):
            o_ref[...] = di * (s_ref[0] + s_ref[1] + g_ref[...]) + b_ref[c]
    sh = jax.ShapeDtypeStruct(g[0].shape, jnp.float32)
    return pl.pallas_call(body, out_shape=(sh, sh, sh))(
        sp[0], sp[1], sp[2], g[0], g[1], g[2], dinv, b)


def kernel(x, edge_index, W1, b1, W2, b2, W3, b3):
    n, d = x.shape
    e = edge_index.shape[1]

    np_ = -(-n // 25600) * 25600
    nf = np_ // 128

    n_chunks = -(-e // (NW * IB * CB))
    epw_rows = n_chunks * CB
    ep = NW * epw_rows * IB

    pad = jnp.full((2, ep - e), n, dtype=edge_index.dtype)
    ei = jnp.concatenate([edge_index, pad], axis=1)
    src2d = ei[0]
    dst2d = ei[1]

    def colpad(v):
        return jnp.zeros((np_,), jnp.float32).at[:n].set(v).reshape(nf, 128)

    xcols = tuple(colpad(x[:, c]) for c in range(d))

    epass = _edge_pass(np_, epw_rows, n_chunks)
    dpass = _deg_pass(np_, epw_rows, n_chunks)

    def flat3(t):
        return tuple(v.reshape(np_,) for v in t)

    def stack3(t):
        return tuple(v.reshape(NC, nf, 128) for v in t)

    degp = dpass(dst2d)
    dinv = _tc_dinv(degp, np_)

    g = _tc_g0(*xcols, dinv, W1)
    sp = stack3(epass(*flat3(g), src2d, dst2d))
    g = _tc_layer(sp, g, dinv, b1, W2)
    sp = stack3(epass(*flat3(g), src2d, dst2d))
    g = _tc_layer(sp, g, dinv, b2, W3)
    sp = stack3(epass(*flat3(g), src2d, dst2d))
    out = _tc_final(sp, g, dinv, b3)
    return jnp.stack([o.reshape(np_)[:n] for o in out], axis=1)
